# baseline (speedup 1.0000x reference)
import functools

import jax
import jax.numpy as jnp
from jax import lax
from jax.experimental import pallas as pl
from jax.experimental.pallas import tpu as pltpu
from jax.experimental.pallas import tpu_sc as plsc

B = 2
N = 1024
E = 16384
F_IN = 64
F_E = 16
CH = 128

NC = 2
NS = 16
NW = NC * NS
CK = 64

_f32 = jnp.float32


@functools.lru_cache(maxsize=None)
def _sc_edge_pass(n_nodes, n_edges, edge_head):
    epw = n_edges // NW
    nchunk = epw // CK
    rpt = n_nodes // NS
    mesh = plsc.VectorSubcoreMesh(core_axis_name="c", subcore_axis_name="s")

    out_type = [
        jax.ShapeDtypeStruct((NC, n_nodes, CH), _f32),
        jax.ShapeDtypeStruct((NC, n_nodes, 16), _f32),
    ]
    if edge_head:
        out_type.append(jax.ShapeDtypeStruct((n_edges, CH), _f32))
    scratch = [
        pltpu.VMEM((nchunk, CK), jnp.int32),
        pltpu.VMEM((nchunk, CK), jnp.int32),
        pltpu.VMEM((2, CK, CH), _f32),
        pltpu.VMEM((2, CK, CH), _f32),
        pltpu.VMEM((2, CK, CH), _f32),
        pltpu.VMEM((CK, 16), _f32),
        pltpu.VMEM_SHARED((n_nodes, CH), _f32),
        pltpu.VMEM_SHARED((n_nodes, 16), _f32),
        pltpu.SemaphoreType.DMA,
        pltpu.SemaphoreType.DMA,
    ]

    def body(*refs):
        it = iter(refs)
        xa, xb, ec, ii, jj = (next(it) for _ in range(5))
        a2t = b2t = ec2 = h2_out = None
        if edge_head:
            a2t, b2t, ec2 = next(it), next(it), next(it)
        h_out, c_out = next(it), next(it)
        if edge_head:
            h2_out = next(it)
        (iv, jv, buf_a, buf_b, buf_e, ones, hacc, cacc,
         sem0, sem1) = (next(it) for _ in range(10))
        cid = lax.axis_index("c")
        sid = lax.axis_index("s")
        wid = sid * NC + cid

        def zrow(r, _):
            for u in range(CH // 16):
                buf_a[0, r, pl.ds(u * 16, 16)] = jnp.zeros((16,), _f32)
            ones[r, pl.ds(0, 16)] = jnp.zeros((16,), _f32)
            return 0
        lax.fori_loop(0, CK, zrow, 0)
        for k in range(rpt // CK):
            off = sid * rpt + k * CK
            pltpu.sync_copy(buf_a.at[0], hacc.at[pl.ds(off, CK)])
            pltpu.sync_copy(ones, cacc.at[pl.ds(off, CK)])

        def orow(r, _):
            ones[r, pl.ds(0, 16)] = jnp.full((16,), 1.0, _f32)
            return 0
        lax.fori_loop(0, CK, orow, 0)
        plsc.subcore_barrier()

        pltpu.sync_copy(ii.at[wid], iv)
        pltpu.sync_copy(jj.at[wid], jv)
        sems = [sem0, sem1]

        def run_pipeline(ta, tb, te, consume):
            for c in range(nchunk):
                s = c % 2
                off = wid * epw + c * CK
                pltpu.sync_copy(ta.at[iv.at[c]], buf_a.at[s])
                pltpu.sync_copy(tb.at[jv.at[c]], buf_b.at[s])
                pltpu.sync_copy(te.at[pl.ds(off, CK)], buf_e.at[s])

                def relu_row(r, _):
                    for u in range(CH // 16):
                        sl = pl.ds(u * 16, 16)
                        v = buf_a[s, r, sl] + buf_b[s, r, sl] + buf_e[s, r, sl]
                        buf_a[s, r, sl] = jnp.maximum(v, 0.0)
                    return 0
                lax.fori_loop(0, CK, relu_row, 0)
                consume(c, s)

        def consume_main(c, s):
            pltpu.sync_copy(buf_a.at[s], hacc.at[iv.at[c]], add=True)
            pltpu.sync_copy(ones, cacc.at[iv.at[c]], add=True)
        run_pipeline(xa, xb, ec, consume_main)
        if edge_head:
            def consume_head(c, s):
                pltpu.sync_copy(buf_a.at[s],
                                h2_out.at[pl.ds(wid * epw + c * CK, CK)])
            run_pipeline(a2t, b2t, ec2, consume_head)

        plsc.subcore_barrier()
        for k in range(rpt // CK):
            off = sid * rpt + k * CK
            pltpu.sync_copy(hacc.at[pl.ds(off, CK)],
                            h_out.at[cid, pl.ds(off, CK)])
            pltpu.sync_copy(cacc.at[pl.ds(off, CK)],
                            c_out.at[cid, pl.ds(off, CK)])

    return pl.kernel(body, out_type=tuple(out_type), mesh=mesh,
                     scratch_types=tuple(scratch))


def _dot(a, b):
    return jnp.dot(a, b, preferred_element_type=_f32)


def _full(shape):
    return pl.BlockSpec(shape, lambda *_: tuple(0 for _ in shape))


def _feat_pre(nodes, wf, bf, w0, b0):
    m = nodes.shape[0]
    bm = 512

    def body(n_ref, wf_ref, bf_ref, w0_ref, b0_ref, x_ref, xa_ref, xb_ref):
        x = jnp.maximum(_dot(n_ref[...], wf_ref[...]) + bf_ref[...], 0.0)
        x_ref[...] = x
        w0v = w0_ref[...]
        xa_ref[...] = _dot(x, w0v[:CH]) + b0_ref[...]
        xb_ref[...] = _dot(x, w0v[CH:2 * CH])

    grid = (m // bm,)
    bs_row = pl.BlockSpec((bm, F_IN), lambda i: (i, 0))
    bs_out = pl.BlockSpec((bm, CH), lambda i: (i, 0))
    return pl.pallas_call(
        body, grid=grid,
        in_specs=[bs_row, _full((F_IN, CH)), _full((1, CH)),
                  _full((2 * CH + F_E, CH)), _full((1, CH))],
        out_specs=[bs_out, bs_out, bs_out],
        out_shape=[jax.ShapeDtypeStruct((m, CH), _f32)] * 3,
    )(nodes, wf, bf, w0, b0)


def _ec1(e, w, bm=4096):
    k, dk = e.shape
    dout = w.shape[1]

    def body(e_ref, w_ref, o_ref):
        o_ref[...] = _dot(e_ref[...], w_ref[...])

    return pl.pallas_call(
        body, grid=(k // bm,),
        in_specs=[pl.BlockSpec((bm, dk), lambda i: (i, 0)),
                  _full((dk, dout))],
        out_specs=pl.BlockSpec((bm, dout), lambda i: (i, 0)),
        out_shape=jax.ShapeDtypeStruct((k, dout), _f32),
    )(e, w)


def _ec2(h2, we1, be1, w0c, bm=4096):
    k, dk = h2.shape

    def body(h_ref, w1_ref, b1_ref, wc_ref, o_ref):
        en = _dot(h_ref[...], w1_ref[...]) + b1_ref[...]
        o_ref[...] = _dot(en, wc_ref[...])

    return pl.pallas_call(
        body, grid=(k // bm,),
        in_specs=[pl.BlockSpec((bm, dk), lambda i: (i, 0)),
                  _full((dk, F_E)), _full((1, F_E)), _full((F_E, CH))],
        out_specs=pl.BlockSpec((bm, CH), lambda i: (i, 0)),
        out_shape=jax.ShapeDtypeStruct((k, CH), _f32),
    )(h2, we1, be1, w0c)


def _post_pre(x, h, cnt, mp, nxt_w0=None, nxt_b0=None, eh_w0=None, eh_b0=None):
    m = x.shape[0]
    bm = 512
    w1m, b1m = mp["msg"][1]
    wu0, bu0 = mp["upd"][0]
    wu1, bu1 = mp["upd"][1]
    b1m, bu0, bu1 = b1m[None], bu0[None], bu1[None]
    has_next = nxt_w0 is not None
    has_eh = eh_w0 is not None

    def body(*refs):
        it = iter(refs)
        x_ref, h_ref, c_ref = next(it), next(it), next(it)
        w1m_ref, b1m_ref = next(it), next(it)
        wu0_ref, bu0_ref, wu1_ref, bu1_ref = (next(it), next(it),
                                              next(it), next(it))
        nw_ref = nb_ref = ew_ref = eb_ref = None
        if has_next:
            nw_ref, nb_ref = next(it), next(it)
        if has_eh:
            ew_ref, eb_ref = next(it), next(it)
        xn_ref = next(it)
        x_val = x_ref[...]
        hs = h_ref[0] + h_ref[1]
        cnts = c_ref[0, :, 0:1] + c_ref[1, :, 0:1]
        agg = _dot(hs, w1m_ref[...]) + cnts * b1m_ref[...]
        wu0v = wu0_ref[...]
        u = jnp.maximum(_dot(x_val, wu0v[:CH]) + _dot(agg, wu0v[CH:])
                        + bu0_ref[...], 0.0)
        xn = x_val + _dot(u, wu1_ref[...]) + bu1_ref[...]
        xn_ref[...] = xn
        if has_next:
            nwv = nw_ref[...]
            next(it)[...] = _dot(xn, nwv[:CH]) + nb_ref[...]
            next(it)[...] = _dot(xn, nwv[CH:2 * CH])
        if has_eh:
            ewv = ew_ref[...]
            next(it)[...] = _dot(xn, ewv[:CH]) + eb_ref[...]
            next(it)[...] = _dot(xn, ewv[CH:2 * CH])

    bs_row = pl.BlockSpec((bm, CH), lambda i: (i, 0))
    ins = [bs_row,
           pl.BlockSpec((NC, bm, CH), lambda i: (0, i, 0)),
           pl.BlockSpec((NC, bm, 16), lambda i: (0, i, 0)),
           _full((CH, CH)), _full((1, CH)),
           _full((2 * CH, CH)), _full((1, CH)),
           _full((CH, CH)), _full((1, CH))]
    args = [x, h, cnt, w1m, b1m, wu0, bu0, wu1, bu1]
    outs = [bs_row]
    out_shape = [jax.ShapeDtypeStruct((m, CH), _f32)]
    if has_next:
        ins += [_full((2 * CH + F_E, CH)), _full((1, CH))]
        args += [nxt_w0, nxt_b0[None]]
        outs += [bs_row, bs_row]
        out_shape += [jax.ShapeDtypeStruct((m, CH), _f32)] * 2
    if has_eh:
        dw_eh = eh_w0.shape[1]
        ins += [_full((2 * CH + F_E, dw_eh)), _full((1, dw_eh))]
        args += [eh_w0, eh_b0[None]]
        bs_eh = pl.BlockSpec((bm, dw_eh), lambda i: (i, 0))
        outs += [bs_eh, bs_eh]
        out_shape += [jax.ShapeDtypeStruct((m, dw_eh), _f32)] * 2
    return pl.pallas_call(
        body, grid=(m // bm,),
        in_specs=ins, out_specs=outs, out_shape=out_shape,
    )(*args)


def _heads(x4, masks, p):
    aw = jnp.stack([p["P_join"][0][0], p["p1_join"][0][0], p["p2_join"][0][0]])
    ab = jnp.stack([p["P_join"][0][1], p["p1_join"][0][1],
                    p["p2_join"][0][1]])[:, None]
    bw = jnp.stack([p["P_join"][1][0], p["p1_join"][1][0], p["p2_join"][1][0]])
    bb = jnp.stack([p["P_join"][1][1], p["p1_join"][1][1],
                    p["p2_join"][1][1]])[:, None]
    cw = jnp.stack([p["P_out"][0][0], p["p1_out"][0][0], p["p2_out"][0][0]])
    cb = jnp.stack([p["P_out"][0][1], p["p1_out"][0][1],
                    p["p2_out"][0][1]])[:, None]
    dw = jnp.stack([p["P_out"][1][0][:, 0], p["p1_out"][1][0][:, 0],
                    p["p2_out"][1][0][:, 0]])[:, None]

    def body(x_ref, m_ref, aw_ref, ab_ref, bw_ref, bb_ref, cw_ref, cb_ref,
             dw_ref, o_ref):
        h = pl.program_id(1)
        xv = x_ref[...]
        pad = m_ref[0, 0, :]
        t = jnp.maximum(_dot(xv, aw_ref[0]) + ab_ref[0], 0.0)
        npx = (_dot(t, bw_ref[0]) + bb_ref[0]) * pad[:, None]
        t2 = jnp.maximum(_dot(npx, cw_ref[0]) + cb_ref[0], 0.0)
        logits = jnp.sum(t2 * dw_ref[0], axis=1)
        z = jnp.exp(logits - jnp.max(logits))
        fm = jnp.where(h == 0, m_ref[0, 1, :], m_ref[0, 2, :])
        o_ref[0, 0, :] = z / jnp.sum(z) * fm

    out = pl.pallas_call(
        body, grid=(B, 3),
        in_specs=[
            pl.BlockSpec((N, CH), lambda b, h: (b * 2 + (1 - h // 2), 0)),
            pl.BlockSpec((1, 3, N), lambda b, h: (b, 0, 1 - h // 2)),
            pl.BlockSpec((1, CH, CH), lambda b, h: (h, 0, 0)),
            pl.BlockSpec((1, 1, CH), lambda b, h: (h, 0, 0)),
            pl.BlockSpec((1, CH, CH), lambda b, h: (h, 0, 0)),
            pl.BlockSpec((1, 1, CH), lambda b, h: (h, 0, 0)),
            pl.BlockSpec((1, CH, CH // 2), lambda b, h: (h, 0, 0)),
            pl.BlockSpec((1, 1, CH // 2), lambda b, h: (h, 0, 0)),
            pl.BlockSpec((1, 1, CH // 2), lambda b, h: (h, 0, 0)),
        ],
        out_specs=pl.BlockSpec((1, 1, N), lambda b, h: (b * 3 + h, 0, 0)),
        out_shape=jax.ShapeDtypeStruct((B * 3, 1, N), _f32),
    )(x4, masks, aw, ab, bw, bb, cw, cb, dw)
    return out.reshape(B, 3, N)


def kernel(masks, nodes, edges, i_s, j_s, params):
    p = params
    m_nodes = 4 * N

    e_intra = edges[:, :2].reshape(4 * E, F_E)
    e_int = edges[:, 2].reshape(2 * E, F_E)
    offs4 = (jnp.arange(4, dtype=jnp.int32) * N)[:, None]
    offs2 = (jnp.arange(2, dtype=jnp.int32) * 2 * N)[:, None]
    ii_a = (i_s[:, :2].reshape(4, E) + offs4).reshape(NW, -1, CK)
    jj_a = (j_s[:, :2].reshape(4, E) + offs4).reshape(NW, -1, CK)
    ii_b = (i_s[:, 2] + offs2).reshape(NW, -1, CK)
    jj_b = (j_s[:, 2] + offs2).reshape(NW, -1, CK)

    sc_intra = _sc_edge_pass(m_nodes, 4 * E, False)
    sc_inter0 = _sc_edge_pass(m_nodes, 2 * E, True)
    sc_inter1 = _sc_edge_pass(m_nodes, 2 * E, False)

    i1_0, i1_1 = p["intra1"][0], p["intra1"][1]
    n0_0, n0_1 = p["inter"][0], p["inter"][1]

    ehw = jnp.pad(n0_0["edge"][0][0], ((0, 0), (0, CH - F_E)))
    ehb = jnp.pad(n0_0["edge"][0][1], (0, CH - F_E))

    ec_a0 = _ec1(e_intra, i1_0["msg"][0][0][2 * CH:])
    ec_a1 = _ec1(e_intra, i1_1["msg"][0][0][2 * CH:])
    ec_b0 = _ec1(e_int, n0_0["msg"][0][0][2 * CH:])
    ec2_b0 = _ec1(e_int, ehw[2 * CH:])

    x0, xa, xb = _feat_pre(nodes.reshape(m_nodes, F_IN),
                           p["feat_in"][0][0], p["feat_in"][0][1][None],
                           i1_0["msg"][0][0], i1_0["msg"][0][1][None])
    h, cnt_a = sc_intra(xa, xb, ec_a0, ii_a, jj_a)
    x1, xa, xb = _post_pre(x0, h, cnt_a, i1_0,
                           i1_1["msg"][0][0], i1_1["msg"][0][1])
    h, cnt_a = sc_intra(xa, xb, ec_a1, ii_a, jj_a)
    x2, xa, xb, a2, b2 = _post_pre(x1, h, cnt_a, i1_1,
                                   n0_0["msg"][0][0], n0_0["msg"][0][1],
                                   ehw, ehb)
    h, cnt_b, h2 = sc_inter0(xa, xb, ec_b0, ii_b, jj_b, a2, b2, ec2_b0)
    x3, xa, xb = _post_pre(x2, h, cnt_b, n0_0,
                           n0_1["msg"][0][0], n0_1["msg"][0][1])
    ec_b1 = _ec2(h2, jnp.pad(n0_0["edge"][1][0], ((0, CH - F_E), (0, 0))),
                 n0_0["edge"][1][1][None], n0_1["msg"][0][0][2 * CH:])
    h, cnt_b = sc_inter1(xa, xb, ec_b1, ii_b, jj_b)
    x4 = _post_pre(x3, h, cnt_b, n0_1)[0]

    return _heads(x4, masks, p)


# speedup vs baseline: 1.3797x; 1.3797x over previous
import functools

import jax
import jax.numpy as jnp
from jax import lax
from jax.experimental import pallas as pl
from jax.experimental.pallas import tpu as pltpu
from jax.experimental.pallas import tpu_sc as plsc

B = 2
N = 1024
E = 16384
F_IN = 64
F_E = 16
CH = 128

NC = 2
NS = 16
NW = NC * NS
CK = 64

_f32 = jnp.float32


@functools.lru_cache(maxsize=None)
def _sc_edge_pass(n_nodes, n_edges, edge_head):
    epw = n_edges // NW
    nchunk = epw // CK
    rpt = n_nodes // NS
    mesh = plsc.VectorSubcoreMesh(core_axis_name="c", subcore_axis_name="s")

    out_type = [
        jax.ShapeDtypeStruct((NC, n_nodes, CH), _f32),
        jax.ShapeDtypeStruct((NC, n_nodes, 16), _f32),
    ]
    if edge_head:
        out_type.append(jax.ShapeDtypeStruct((n_edges, CH), _f32))
    scratch = [
        pltpu.VMEM((nchunk, CK), jnp.int32),
        pltpu.VMEM((nchunk, CK), jnp.int32),
        pltpu.VMEM((2, CK, CH), _f32),
        pltpu.VMEM((2, CK, CH), _f32),
        pltpu.VMEM((2, CK, CH), _f32),
        pltpu.VMEM((CK, 16), _f32),
        pltpu.VMEM_SHARED((n_nodes, CH), _f32),
        pltpu.VMEM_SHARED((n_nodes, 16), _f32),
        pltpu.SemaphoreType.DMA,
        pltpu.SemaphoreType.DMA,
        pltpu.SemaphoreType.DMA,
        pltpu.SemaphoreType.DMA,
        pltpu.SemaphoreType.DMA,
        pltpu.SemaphoreType.DMA,
    ]

    def body(*refs):
        it = iter(refs)
        xa, xb, ec, ii, jj = (next(it) for _ in range(5))
        a2t = b2t = ec2 = h2_out = None
        if edge_head:
            a2t, b2t, ec2 = next(it), next(it), next(it)
        h_out, c_out = next(it), next(it)
        if edge_head:
            h2_out = next(it)
        (iv, jv, buf_a, buf_b, buf_e, ones, hacc, cacc,
         sa0, sa1, sb0, sb1, se0, se1) = (next(it) for _ in range(14))
        cid = lax.axis_index("c")
        sid = lax.axis_index("s")
        wid = sid * NC + cid

        def zrow(r, _):
            for u in range(CH // 16):
                buf_a[0, r, pl.ds(u * 16, 16)] = jnp.zeros((16,), _f32)
            ones[r, pl.ds(0, 16)] = jnp.zeros((16,), _f32)
            return 0
        lax.fori_loop(0, CK, zrow, 0)
        for k in range(rpt // CK):
            off = sid * rpt + k * CK
            pltpu.sync_copy(buf_a.at[0], hacc.at[pl.ds(off, CK)])
            pltpu.sync_copy(ones, cacc.at[pl.ds(off, CK)])

        def orow(r, _):
            ones[r, pl.ds(0, 16)] = jnp.full((16,), 1.0, _f32)
            return 0
        lax.fori_loop(0, CK, orow, 0)
        plsc.subcore_barrier()

        pltpu.sync_copy(ii.at[wid], iv)
        pltpu.sync_copy(jj.at[wid], jv)
        sems_a, sems_b, sems_e = [sa0, sa1], [sb0, sb1], [se0, se1]

        def run_pipeline(ta, tb, te, consume):
            def issue(c):
                s = c % 2
                off = wid * epw + c * CK
                return [
                    pltpu.async_copy(ta.at[iv.at[c]], buf_a.at[s], sems_a[s]),
                    pltpu.async_copy(tb.at[jv.at[c]], buf_b.at[s], sems_b[s]),
                    pltpu.async_copy(te.at[pl.ds(off, CK)], buf_e.at[s],
                                     sems_e[s]),
                ]
            desc = {0: issue(0)}
            for c in range(nchunk):
                for d in desc.pop(c % 2):
                    d.wait()
                s = c % 2

                def relu_row(r, _):
                    for u in range(CH // 16):
                        sl = pl.ds(u * 16, 16)
                        v = buf_a[s, r, sl] + buf_b[s, r, sl] + buf_e[s, r, sl]
                        buf_a[s, r, sl] = jnp.maximum(v, 0.0)
                    return 0
                lax.fori_loop(0, CK, relu_row, 0)
                if c + 1 < nchunk:
                    desc[(c + 1) % 2] = issue(c + 1)
                consume(c, s)

        def consume_main(c, s):
            pltpu.sync_copy(buf_a.at[s], hacc.at[iv.at[c]], add=True)
            pltpu.sync_copy(ones, cacc.at[iv.at[c]], add=True)
        run_pipeline(xa, xb, ec, consume_main)
        if edge_head:
            def consume_head(c, s):
                pltpu.sync_copy(buf_a.at[s],
                                h2_out.at[pl.ds(wid * epw + c * CK, CK)])
            run_pipeline(a2t, b2t, ec2, consume_head)

        plsc.subcore_barrier()
        for k in range(rpt // CK):
            off = sid * rpt + k * CK
            pltpu.sync_copy(hacc.at[pl.ds(off, CK)],
                            h_out.at[cid, pl.ds(off, CK)])
            pltpu.sync_copy(cacc.at[pl.ds(off, CK)],
                            c_out.at[cid, pl.ds(off, CK)])

    return pl.kernel(body, out_type=tuple(out_type), mesh=mesh,
                     scratch_types=tuple(scratch))


def _dot(a, b):
    return jnp.dot(a, b, preferred_element_type=_f32)


def _full(shape):
    return pl.BlockSpec(shape, lambda *_: tuple(0 for _ in shape))


def _feat_pre(nodes, wf, bf, w0, b0):
    m = nodes.shape[0]
    bm = 512

    def body(n_ref, wf_ref, bf_ref, w0_ref, b0_ref, x_ref, xa_ref, xb_ref):
        x = jnp.maximum(_dot(n_ref[...], wf_ref[...]) + bf_ref[...], 0.0)
        x_ref[...] = x
        w0v = w0_ref[...]
        xa_ref[...] = _dot(x, w0v[:CH]) + b0_ref[...]
        xb_ref[...] = _dot(x, w0v[CH:2 * CH])

    grid = (m // bm,)
    bs_row = pl.BlockSpec((bm, F_IN), lambda i: (i, 0))
    bs_out = pl.BlockSpec((bm, CH), lambda i: (i, 0))
    return pl.pallas_call(
        body, grid=grid,
        in_specs=[bs_row, _full((F_IN, CH)), _full((1, CH)),
                  _full((2 * CH + F_E, CH)), _full((1, CH))],
        out_specs=[bs_out, bs_out, bs_out],
        out_shape=[jax.ShapeDtypeStruct((m, CH), _f32)] * 3,
    )(nodes, wf, bf, w0, b0)


def _ec1(e, w, bm=4096):
    k, dk = e.shape
    dout = w.shape[1]

    def body(e_ref, w_ref, o_ref):
        o_ref[...] = _dot(e_ref[...], w_ref[...])

    return pl.pallas_call(
        body, grid=(k // bm,),
        in_specs=[pl.BlockSpec((bm, dk), lambda i: (i, 0)),
                  _full((dk, dout))],
        out_specs=pl.BlockSpec((bm, dout), lambda i: (i, 0)),
        out_shape=jax.ShapeDtypeStruct((k, dout), _f32),
    )(e, w)


def _ec2(h2, we1, be1, w0c, bm=4096):
    k, dk = h2.shape

    def body(h_ref, w1_ref, b1_ref, wc_ref, o_ref):
        en = _dot(h_ref[...], w1_ref[...]) + b1_ref[...]
        o_ref[...] = _dot(en, wc_ref[...])

    return pl.pallas_call(
        body, grid=(k // bm,),
        in_specs=[pl.BlockSpec((bm, dk), lambda i: (i, 0)),
                  _full((dk, F_E)), _full((1, F_E)), _full((F_E, CH))],
        out_specs=pl.BlockSpec((bm, CH), lambda i: (i, 0)),
        out_shape=jax.ShapeDtypeStruct((k, CH), _f32),
    )(h2, we1, be1, w0c)


def _post_pre(x, h, cnt, mp, nxt_w0=None, nxt_b0=None, eh_w0=None, eh_b0=None):
    m = x.shape[0]
    bm = 512
    w1m, b1m = mp["msg"][1]
    wu0, bu0 = mp["upd"][0]
    wu1, bu1 = mp["upd"][1]
    b1m, bu0, bu1 = b1m[None], bu0[None], bu1[None]
    has_next = nxt_w0 is not None
    has_eh = eh_w0 is not None

    def body(*refs):
        it = iter(refs)
        x_ref, h_ref, c_ref = next(it), next(it), next(it)
        w1m_ref, b1m_ref = next(it), next(it)
        wu0_ref, bu0_ref, wu1_ref, bu1_ref = (next(it), next(it),
                                              next(it), next(it))
        nw_ref = nb_ref = ew_ref = eb_ref = None
        if has_next:
            nw_ref, nb_ref = next(it), next(it)
        if has_eh:
            ew_ref, eb_ref = next(it), next(it)
        xn_ref = next(it)
        x_val = x_ref[...]
        hs = h_ref[0] + h_ref[1]
        cnts = c_ref[0, :, 0:1] + c_ref[1, :, 0:1]
        agg = _dot(hs, w1m_ref[...]) + cnts * b1m_ref[...]
        wu0v = wu0_ref[...]
        u = jnp.maximum(_dot(x_val, wu0v[:CH]) + _dot(agg, wu0v[CH:])
                        + bu0_ref[...], 0.0)
        xn = x_val + _dot(u, wu1_ref[...]) + bu1_ref[...]
        xn_ref[...] = xn
        if has_next:
            nwv = nw_ref[...]
            next(it)[...] = _dot(xn, nwv[:CH]) + nb_ref[...]
            next(it)[...] = _dot(xn, nwv[CH:2 * CH])
        if has_eh:
            ewv = ew_ref[...]
            next(it)[...] = _dot(xn, ewv[:CH]) + eb_ref[...]
            next(it)[...] = _dot(xn, ewv[CH:2 * CH])

    bs_row = pl.BlockSpec((bm, CH), lambda i: (i, 0))
    ins = [bs_row,
           pl.BlockSpec((NC, bm, CH), lambda i: (0, i, 0)),
           pl.BlockSpec((NC, bm, 16), lambda i: (0, i, 0)),
           _full((CH, CH)), _full((1, CH)),
           _full((2 * CH, CH)), _full((1, CH)),
           _full((CH, CH)), _full((1, CH))]
    args = [x, h, cnt, w1m, b1m, wu0, bu0, wu1, bu1]
    outs = [bs_row]
    out_shape = [jax.ShapeDtypeStruct((m, CH), _f32)]
    if has_next:
        ins += [_full((2 * CH + F_E, CH)), _full((1, CH))]
        args += [nxt_w0, nxt_b0[None]]
        outs += [bs_row, bs_row]
        out_shape += [jax.ShapeDtypeStruct((m, CH), _f32)] * 2
    if has_eh:
        dw_eh = eh_w0.shape[1]
        ins += [_full((2 * CH + F_E, dw_eh)), _full((1, dw_eh))]
        args += [eh_w0, eh_b0[None]]
        bs_eh = pl.BlockSpec((bm, dw_eh), lambda i: (i, 0))
        outs += [bs_eh, bs_eh]
        out_shape += [jax.ShapeDtypeStruct((m, dw_eh), _f32)] * 2
    return pl.pallas_call(
        body, grid=(m // bm,),
        in_specs=ins, out_specs=outs, out_shape=out_shape,
    )(*args)


def _heads(x4, masks, p):
    aw = jnp.stack([p["P_join"][0][0], p["p1_join"][0][0], p["p2_join"][0][0]])
    ab = jnp.stack([p["P_join"][0][1], p["p1_join"][0][1],
                    p["p2_join"][0][1]])[:, None]
    bw = jnp.stack([p["P_join"][1][0], p["p1_join"][1][0], p["p2_join"][1][0]])
    bb = jnp.stack([p["P_join"][1][1], p["p1_join"][1][1],
                    p["p2_join"][1][1]])[:, None]
    cw = jnp.stack([p["P_out"][0][0], p["p1_out"][0][0], p["p2_out"][0][0]])
    cb = jnp.stack([p["P_out"][0][1], p["p1_out"][0][1],
                    p["p2_out"][0][1]])[:, None]
    dw = jnp.stack([p["P_out"][1][0][:, 0], p["p1_out"][1][0][:, 0],
                    p["p2_out"][1][0][:, 0]])[:, None]

    def body(x_ref, m_ref, aw_ref, ab_ref, bw_ref, bb_ref, cw_ref, cb_ref,
             dw_ref, o_ref):
        h = pl.program_id(1)
        xv = x_ref[...]
        pad = m_ref[0, 0, :]
        t = jnp.maximum(_dot(xv, aw_ref[0]) + ab_ref[0], 0.0)
        npx = (_dot(t, bw_ref[0]) + bb_ref[0]) * pad[:, None]
        t2 = jnp.maximum(_dot(npx, cw_ref[0]) + cb_ref[0], 0.0)
        logits = jnp.sum(t2 * dw_ref[0], axis=1)
        z = jnp.exp(logits - jnp.max(logits))
        fm = jnp.where(h == 0, m_ref[0, 1, :], m_ref[0, 2, :])
        o_ref[0, 0, :] = z / jnp.sum(z) * fm

    out = pl.pallas_call(
        body, grid=(B, 3),
        in_specs=[
            pl.BlockSpec((N, CH), lambda b, h: (b * 2 + (1 - h // 2), 0)),
            pl.BlockSpec((1, 3, N), lambda b, h: (b, 0, 1 - h // 2)),
            pl.BlockSpec((1, CH, CH), lambda b, h: (h, 0, 0)),
            pl.BlockSpec((1, 1, CH), lambda b, h: (h, 0, 0)),
            pl.BlockSpec((1, CH, CH), lambda b, h: (h, 0, 0)),
            pl.BlockSpec((1, 1, CH), lambda b, h: (h, 0, 0)),
            pl.BlockSpec((1, CH, CH // 2), lambda b, h: (h, 0, 0)),
            pl.BlockSpec((1, 1, CH // 2), lambda b, h: (h, 0, 0)),
            pl.BlockSpec((1, 1, CH // 2), lambda b, h: (h, 0, 0)),
        ],
        out_specs=pl.BlockSpec((1, 1, N), lambda b, h: (b * 3 + h, 0, 0)),
        out_shape=jax.ShapeDtypeStruct((B * 3, 1, N), _f32),
    )(x4, masks, aw, ab, bw, bb, cw, cb, dw)
    return out.reshape(B, 3, N)


def kernel(masks, nodes, edges, i_s, j_s, params):
    p = params
    m_nodes = 4 * N

    e_intra = edges[:, :2].reshape(4 * E, F_E)
    e_int = edges[:, 2].reshape(2 * E, F_E)
    offs4 = (jnp.arange(4, dtype=jnp.int32) * N)[:, None]
    offs2 = (jnp.arange(2, dtype=jnp.int32) * 2 * N)[:, None]
    ii_a = (i_s[:, :2].reshape(4, E) + offs4).reshape(NW, -1, CK)
    jj_a = (j_s[:, :2].reshape(4, E) + offs4).reshape(NW, -1, CK)
    ii_b = (i_s[:, 2] + offs2).reshape(NW, -1, CK)
    jj_b = (j_s[:, 2] + offs2).reshape(NW, -1, CK)

    sc_intra = _sc_edge_pass(m_nodes, 4 * E, False)
    sc_inter0 = _sc_edge_pass(m_nodes, 2 * E, True)
    sc_inter1 = _sc_edge_pass(m_nodes, 2 * E, False)

    i1_0, i1_1 = p["intra1"][0], p["intra1"][1]
    n0_0, n0_1 = p["inter"][0], p["inter"][1]

    ehw = jnp.pad(n0_0["edge"][0][0], ((0, 0), (0, CH - F_E)))
    ehb = jnp.pad(n0_0["edge"][0][1], (0, CH - F_E))

    ec_a0 = _ec1(e_intra, i1_0["msg"][0][0][2 * CH:])
    ec_a1 = _ec1(e_intra, i1_1["msg"][0][0][2 * CH:])
    ec_b0 = _ec1(e_int, n0_0["msg"][0][0][2 * CH:])
    ec2_b0 = _ec1(e_int, ehw[2 * CH:])

    x0, xa, xb = _feat_pre(nodes.reshape(m_nodes, F_IN),
                           p["feat_in"][0][0], p["feat_in"][0][1][None],
                           i1_0["msg"][0][0], i1_0["msg"][0][1][None])
    h, cnt_a = sc_intra(xa, xb, ec_a0, ii_a, jj_a)
    x1, xa, xb = _post_pre(x0, h, cnt_a, i1_0,
                           i1_1["msg"][0][0], i1_1["msg"][0][1])
    h, cnt_a = sc_intra(xa, xb, ec_a1, ii_a, jj_a)
    x2, xa, xb, a2, b2 = _post_pre(x1, h, cnt_a, i1_1,
                                   n0_0["msg"][0][0], n0_0["msg"][0][1],
                                   ehw, ehb)
    h, cnt_b, h2 = sc_inter0(xa, xb, ec_b0, ii_b, jj_b, a2, b2, ec2_b0)
    x3, xa, xb = _post_pre(x2, h, cnt_b, n0_0,
                           n0_1["msg"][0][0], n0_1["msg"][0][1])
    ec_b1 = _ec2(h2, jnp.pad(n0_0["edge"][1][0], ((0, CH - F_E), (0, 0))),
                 n0_0["edge"][1][1][None], n0_1["msg"][0][0][2 * CH:])
    h, cnt_b = sc_inter1(xa, xb, ec_b1, ii_b, jj_b)
    x4 = _post_pre(x3, h, cnt_b, n0_1)[0]

    return _heads(x4, masks, p)
